# Initial kernel scaffold; baseline (speedup 1.0000x reference)
import jax
import jax.numpy as jnp
from jax.experimental import pallas as pl


def kernel(x, edge_index, Wl0, bl0, Wr0, g0, be0, rm0, rv0, Wl1, bl1, Wr1, g1, be1, rm1, rv1, Wl2, bl2, Wr2):
    raise NotImplementedError("write your pallas kernel here")


# speedup vs baseline: 4.4663x; 4.4663x over previous
import functools

import jax
import jax.numpy as jnp
from jax import lax
from jax.experimental import pallas as pl
from jax.experimental.pallas import tpu as pltpu
from jax.experimental.pallas import tpu_sc as plsc

N = 10000
D = 128
EPS = 1e-5

NC = 2
NS = 16
NW = NC * NS
CHUNK = 128
NP = 10240
ROWS_PER_TILE = NP // NS
DEGW = 128

BLK = 512
GRID = NP // BLK

_F32 = jnp.float32


def _make_sc_agg(C):
    mesh = plsc.VectorSubcoreMesh(core_axis_name="c", subcore_axis_name="s")

    def body(m_hbm, src_hbm, dst_hbm, acc_out, src_v, dst_v, rows_v,
             zrow_v, acc_sh):
        cid = lax.axis_index("c")
        sid = lax.axis_index("s")
        wid = cid * NS + sid

        def zfill(i, _):
            r = i // (D // 16)
            c = i % (D // 16)
            zrow_v[r, pl.ds(c * 16, 16)] = jnp.zeros((16,), _F32)
            return 0
        lax.fori_loop(0, 16 * (D // 16), zfill, 0)

        base = sid * ROWS_PER_TILE
        def zcopy(i, _):
            pltpu.sync_copy(zrow_v, acc_sh.at[pl.ds(base + i * 16, 16)])
            return 0
        lax.fori_loop(0, ROWS_PER_TILE // 16, zcopy, 0)

        pltpu.sync_copy(src_hbm.at[wid], src_v)
        pltpu.sync_copy(dst_hbm.at[wid], dst_v)

        plsc.subcore_barrier()

        def chunk(j, _):
            pltpu.sync_copy(m_hbm.at[src_v.at[j]], rows_v)
            pltpu.sync_copy(rows_v, acc_sh.at[dst_v.at[j]], add=True)
            return 0
        lax.fori_loop(0, C, chunk, 0)

        plsc.subcore_barrier()

        pltpu.sync_copy(acc_sh.at[pl.ds(base, ROWS_PER_TILE)],
                        acc_out.at[cid, pl.ds(base, ROWS_PER_TILE)])

    return pl.kernel(
        body,
        out_type=[jax.ShapeDtypeStruct((NC, NP, D), _F32)],
        mesh=mesh,
        scratch_types=[
            pltpu.VMEM((C, CHUNK), jnp.int32),
            pltpu.VMEM((C, CHUNK), jnp.int32),
            pltpu.VMEM((CHUNK, D), _F32),
            pltpu.VMEM((16, D), _F32),
            pltpu.VMEM_SHARED((NP, D), _F32),
        ])


def _make_sc_deg(C):
    mesh = plsc.VectorSubcoreMesh(core_axis_name="c", subcore_axis_name="s")

    def body(dst_hbm, deg_out, dst_v, ones_v, zdeg_v, deg_sh):
        cid = lax.axis_index("c")
        sid = lax.axis_index("s")
        wid = cid * NS + sid

        def ofill(i, _):
            ones_v[i, pl.ds(0, DEGW)] = jnp.ones((DEGW,), _F32)
            zdeg_v[i % 16, pl.ds(0, DEGW)] = jnp.zeros((DEGW,), _F32)
            return 0
        lax.fori_loop(0, CHUNK, ofill, 0)

        base = sid * ROWS_PER_TILE
        def zcopy(i, _):
            pltpu.sync_copy(zdeg_v, deg_sh.at[pl.ds(base + i * 16, 16)])
            return 0
        lax.fori_loop(0, ROWS_PER_TILE // 16, zcopy, 0)

        pltpu.sync_copy(dst_hbm.at[wid], dst_v)
        plsc.subcore_barrier()

        def chunk(j, _):
            pltpu.sync_copy(ones_v, deg_sh.at[dst_v.at[j]], add=True)
            return 0
        lax.fori_loop(0, C, chunk, 0)

        plsc.subcore_barrier()
        pltpu.sync_copy(deg_sh.at[pl.ds(base, ROWS_PER_TILE)],
                        deg_out.at[cid, pl.ds(base, ROWS_PER_TILE)])

    return pl.kernel(
        body,
        out_type=[jax.ShapeDtypeStruct((NC, NP, DEGW), _F32)],
        mesh=mesh,
        scratch_types=[
            pltpu.VMEM((C, CHUNK), jnp.int32),
            pltpu.VMEM((CHUNK, DEGW), _F32),
            pltpu.VMEM((16, DEGW), _F32),
            pltpu.VMEM_SHARED((NP, DEGW), _F32),
        ])


def _dotT(a, w):
    return lax.dot_general(a, w, (((1,), (1,)), ((), ())),
                           preferred_element_type=_F32)


def _row_spec():
    return pl.BlockSpec((BLK, D), lambda i: (i, 0))


def _deg_spec():
    return pl.BlockSpec((BLK, DEGW), lambda i: (i, 0))


def _full_spec(shape):
    return pl.BlockSpec(shape, lambda i: tuple(0 for _ in shape))


def _tc_in(x, Wl, Wr, bl):
    def body(x_ref, wl_ref, wr_ref, bl_ref, m_ref, z_ref):
        xv = x_ref[...]
        m_ref[...] = _dotT(xv, wl_ref[...])
        z_ref[...] = _dotT(xv, wr_ref[...]) + bl_ref[...]
    return pl.pallas_call(
        body,
        grid=(GRID,),
        in_specs=[_row_spec(), _full_spec((D, D)), _full_spec((D, D)),
                  _full_spec((1, D))],
        out_specs=[_row_spec(), _row_spec()],
        out_shape=[jax.ShapeDtypeStruct((NP, D), _F32),
                   jax.ShapeDtypeStruct((NP, D), _F32)],
    )(x, Wl, Wr, bl)


def _tc_mid(acc0, acc1, deg0, deg1, z, g, be, rm, rv, Wl, Wr, bl):
    def body(a0, a1, d0, d1, z_ref, g_ref, be_ref, rm_ref, rv_ref,
             wl_ref, wr_ref, bl_ref, m_ref, z2_ref):
        deg = d0[:, 0:1] + d1[:, 0:1]
        inv = 1.0 / jnp.maximum(deg, 1.0)
        s = (a0[...] + a1[...]) * inv + z_ref[...]
        scale = g_ref[...] * lax.rsqrt(rv_ref[...] + EPS)
        h = jnp.maximum((s - rm_ref[...]) * scale + be_ref[...], 0.0)
        m_ref[...] = _dotT(h, wl_ref[...])
        z2_ref[...] = _dotT(h, wr_ref[...]) + bl_ref[...]
    return pl.pallas_call(
        body,
        grid=(GRID,),
        in_specs=[_row_spec(), _row_spec(), _deg_spec(), _deg_spec(),
                  _row_spec(),
                  _full_spec((1, D)), _full_spec((1, D)), _full_spec((1, D)),
                  _full_spec((1, D)),
                  _full_spec((D, D)), _full_spec((D, D)), _full_spec((1, D))],
        out_specs=[_row_spec(), _row_spec()],
        out_shape=[jax.ShapeDtypeStruct((NP, D), _F32),
                   jax.ShapeDtypeStruct((NP, D), _F32)],
    )(acc0, acc1, deg0, deg1, z, g, be, rm, rv, Wl, Wr, bl)


def _tc_out(acc0, acc1, deg0, deg1, z):
    def body(a0, a1, d0, d1, z_ref, o_ref):
        deg = d0[:, 0:1] + d1[:, 0:1]
        inv = 1.0 / jnp.maximum(deg, 1.0)
        o_ref[...] = (a0[...] + a1[...]) * inv + z_ref[...]
    return pl.pallas_call(
        body,
        grid=(GRID,),
        in_specs=[_row_spec(), _row_spec(), _deg_spec(), _deg_spec(),
                  _row_spec()],
        out_specs=_row_spec(),
        out_shape=jax.ShapeDtypeStruct((NP, D), _F32),
    )(acc0, acc1, deg0, deg1, z)


def kernel(x, edge_index, Wl0, bl0, Wr0, g0, be0, rm0, rv0,
           Wl1, bl1, Wr1, g1, be1, rm1, rv1, Wl2, bl2, Wr2):
    E = edge_index.shape[1]
    C = -(-E // (NW * CHUNK))
    e_pad = NW * C * CHUNK - E

    src = edge_index[0]
    dst = edge_index[1]
    src_p = jnp.concatenate([src, jnp.zeros((e_pad,), jnp.int32)])
    dst_p = jnp.concatenate([dst, jnp.full((e_pad,), N, jnp.int32)])
    src_p = src_p.reshape(NW, C, CHUNK)
    dst_p = dst_p.reshape(NW, C, CHUNK)

    x_pad = jnp.concatenate([x, jnp.zeros((NP - N, D), _F32)])

    r1 = lambda v: v.reshape(1, D)

    sc_agg = _make_sc_agg(C)
    sc_deg = _make_sc_deg(C)

    m0, z0 = _tc_in(x_pad, Wl0, Wr0, r1(bl0))
    (deg,) = sc_deg(dst_p)
    (acc,) = sc_agg(m0, src_p, dst_p)
    m1, z1 = _tc_mid(acc[0], acc[1], deg[0], deg[1], z0,
                     r1(g0), r1(be0), r1(rm0), r1(rv0), Wl1, Wr1, r1(bl1))
    (acc,) = sc_agg(m1, src_p, dst_p)
    m2, z2 = _tc_mid(acc[0], acc[1], deg[0], deg[1], z1,
                     r1(g1), r1(be1), r1(rm1), r1(rv1), Wl2, Wr2, r1(bl2))
    (acc,) = sc_agg(m2, src_p, dst_p)
    out = _tc_out(acc[0], acc[1], deg[0], deg[1], z2)
    return out[:N]


# speedup vs baseline: 5.3145x; 1.1899x over previous
import functools

import jax
import jax.numpy as jnp
from jax import lax
from jax.experimental import pallas as pl
from jax.experimental.pallas import tpu as pltpu
from jax.experimental.pallas import tpu_sc as plsc

N = 10000
D = 128
EPS = 1e-5

NC = 2
NS = 16
NW = NC * NS
CHUNK = 128
NP = 10240
ROWS_PER_TILE = NP // NS
DEGW = 128

BLK = 512
GRID = NP // BLK

_F32 = jnp.float32


def _make_sc_agg(C):
    mesh = plsc.VectorSubcoreMesh(core_axis_name="c", subcore_axis_name="s")
    ph0 = (C + 1) // 2
    phases = [(0, ph0), (ph0, C - ph0)]

    def body(m_hbm, src_hbm, dst_hbm, acc_out, src_v, dst_v, rows_v,
             zrow_v, acc_sh, gsem):
        cid = lax.axis_index("c")
        sid = lax.axis_index("s")
        wid = cid * NS + sid

        def zfill(i, _):
            r = i // (D // 16)
            c = i % (D // 16)
            zrow_v[r, pl.ds(c * 16, 16)] = jnp.zeros((16,), _F32)
            return 0
        lax.fori_loop(0, 16 * (D // 16), zfill, 0)

        base = sid * ROWS_PER_TILE
        def zcopy(i, _):
            pltpu.sync_copy(zrow_v, acc_sh.at[pl.ds(base + i * 16, 16)])
            return 0
        lax.fori_loop(0, ROWS_PER_TILE // 16, zcopy, 0)

        plsc.subcore_barrier()

        for off, cnt in phases:
            pltpu.sync_copy(src_hbm.at[wid, pl.ds(off, cnt)],
                            src_v.at[pl.ds(0, cnt)])
            pltpu.sync_copy(dst_hbm.at[wid, pl.ds(off, cnt)],
                            dst_v.at[pl.ds(0, cnt)])
            for b in range(2):
                pltpu.async_copy(m_hbm.at[src_v.at[b]], rows_v.at[b], gsem)

            def chunk(j, _):
                b = j % 2
                pltpu.make_async_copy(m_hbm.at[src_v.at[j]],
                                      rows_v.at[b], gsem).wait()
                pltpu.sync_copy(rows_v.at[b], acc_sh.at[dst_v.at[j]],
                                add=True)

                @pl.when(j + 2 < cnt)
                def _():
                    pltpu.async_copy(m_hbm.at[src_v.at[j + 2]],
                                     rows_v.at[b], gsem)
                return 0
            lax.fori_loop(0, cnt, chunk, 0)

        plsc.subcore_barrier()

        pltpu.sync_copy(acc_sh.at[pl.ds(base, ROWS_PER_TILE)],
                        acc_out.at[cid, pl.ds(base, ROWS_PER_TILE)])

    return pl.kernel(
        body,
        out_type=[jax.ShapeDtypeStruct((NC, NP, D), _F32)],
        mesh=mesh,
        scratch_types=[
            pltpu.VMEM((ph0, CHUNK), jnp.int32),
            pltpu.VMEM((ph0, CHUNK), jnp.int32),
            pltpu.VMEM((2, CHUNK, D), _F32),
            pltpu.VMEM((16, D), _F32),
            pltpu.VMEM_SHARED((NP, D), _F32),
            pltpu.SemaphoreType.DMA,
        ])


def _make_sc_deg(C):
    mesh = plsc.VectorSubcoreMesh(core_axis_name="c", subcore_axis_name="s")

    def body(dst_hbm, deg_out, dst_v, ones_v, zdeg_v, deg_sh):
        cid = lax.axis_index("c")
        sid = lax.axis_index("s")
        wid = cid * NS + sid

        def ofill(i, _):
            ones_v[i, pl.ds(0, DEGW)] = jnp.ones((DEGW,), _F32)
            zdeg_v[i % 16, pl.ds(0, DEGW)] = jnp.zeros((DEGW,), _F32)
            return 0
        lax.fori_loop(0, CHUNK, ofill, 0)

        base = sid * ROWS_PER_TILE
        def zcopy(i, _):
            pltpu.sync_copy(zdeg_v, deg_sh.at[pl.ds(base + i * 16, 16)])
            return 0
        lax.fori_loop(0, ROWS_PER_TILE // 16, zcopy, 0)

        pltpu.sync_copy(dst_hbm.at[wid], dst_v)
        plsc.subcore_barrier()

        def chunk(j, _):
            pltpu.sync_copy(ones_v, deg_sh.at[dst_v.at[j]], add=True)
            return 0
        lax.fori_loop(0, C, chunk, 0)

        plsc.subcore_barrier()
        pltpu.sync_copy(deg_sh.at[pl.ds(base, ROWS_PER_TILE)],
                        deg_out.at[cid, pl.ds(base, ROWS_PER_TILE)])

    return pl.kernel(
        body,
        out_type=[jax.ShapeDtypeStruct((NC, NP, DEGW), _F32)],
        mesh=mesh,
        scratch_types=[
            pltpu.VMEM((C, CHUNK), jnp.int32),
            pltpu.VMEM((CHUNK, DEGW), _F32),
            pltpu.VMEM((16, DEGW), _F32),
            pltpu.VMEM_SHARED((NP, DEGW), _F32),
        ])


def _dotT(a, w):
    return lax.dot_general(a, w, (((1,), (1,)), ((), ())),
                           preferred_element_type=_F32)


def _row_spec():
    return pl.BlockSpec((BLK, D), lambda i: (i, 0))


def _deg_spec():
    return pl.BlockSpec((BLK, DEGW), lambda i: (i, 0))


def _full_spec(shape):
    return pl.BlockSpec(shape, lambda i: tuple(0 for _ in shape))


def _tc_in(x, Wl, Wr, bl):
    def body(x_ref, wl_ref, wr_ref, bl_ref, m_ref, z_ref):
        xv = x_ref[...]
        m_ref[...] = _dotT(xv, wl_ref[...])
        z_ref[...] = _dotT(xv, wr_ref[...]) + bl_ref[...]
    return pl.pallas_call(
        body,
        grid=(GRID,),
        in_specs=[_row_spec(), _full_spec((D, D)), _full_spec((D, D)),
                  _full_spec((1, D))],
        out_specs=[_row_spec(), _row_spec()],
        out_shape=[jax.ShapeDtypeStruct((NP, D), _F32),
                   jax.ShapeDtypeStruct((NP, D), _F32)],
    )(x, Wl, Wr, bl)


def _tc_mid(acc0, acc1, deg0, deg1, z, g, be, rm, rv, Wl, Wr, bl):
    def body(a0, a1, d0, d1, z_ref, g_ref, be_ref, rm_ref, rv_ref,
             wl_ref, wr_ref, bl_ref, m_ref, z2_ref):
        deg = d0[:, 0:1] + d1[:, 0:1]
        inv = 1.0 / jnp.maximum(deg, 1.0)
        s = (a0[...] + a1[...]) * inv + z_ref[...]
        scale = g_ref[...] * lax.rsqrt(rv_ref[...] + EPS)
        h = jnp.maximum((s - rm_ref[...]) * scale + be_ref[...], 0.0)
        m_ref[...] = _dotT(h, wl_ref[...])
        z2_ref[...] = _dotT(h, wr_ref[...]) + bl_ref[...]
    return pl.pallas_call(
        body,
        grid=(GRID,),
        in_specs=[_row_spec(), _row_spec(), _deg_spec(), _deg_spec(),
                  _row_spec(),
                  _full_spec((1, D)), _full_spec((1, D)), _full_spec((1, D)),
                  _full_spec((1, D)),
                  _full_spec((D, D)), _full_spec((D, D)), _full_spec((1, D))],
        out_specs=[_row_spec(), _row_spec()],
        out_shape=[jax.ShapeDtypeStruct((NP, D), _F32),
                   jax.ShapeDtypeStruct((NP, D), _F32)],
    )(acc0, acc1, deg0, deg1, z, g, be, rm, rv, Wl, Wr, bl)


def _tc_out(acc0, acc1, deg0, deg1, z):
    def body(a0, a1, d0, d1, z_ref, o_ref):
        deg = d0[:, 0:1] + d1[:, 0:1]
        inv = 1.0 / jnp.maximum(deg, 1.0)
        o_ref[...] = (a0[...] + a1[...]) * inv + z_ref[...]
    return pl.pallas_call(
        body,
        grid=(GRID,),
        in_specs=[_row_spec(), _row_spec(), _deg_spec(), _deg_spec(),
                  _row_spec()],
        out_specs=_row_spec(),
        out_shape=jax.ShapeDtypeStruct((NP, D), _F32),
    )(acc0, acc1, deg0, deg1, z)


def kernel(x, edge_index, Wl0, bl0, Wr0, g0, be0, rm0, rv0,
           Wl1, bl1, Wr1, g1, be1, rm1, rv1, Wl2, bl2, Wr2):
    E = edge_index.shape[1]
    C = -(-E // (NW * CHUNK))
    e_pad = NW * C * CHUNK - E

    src = edge_index[0]
    dst = edge_index[1]
    src_p = jnp.concatenate([src, jnp.zeros((e_pad,), jnp.int32)])
    dst_p = jnp.concatenate([dst, jnp.full((e_pad,), N, jnp.int32)])
    src_p = src_p.reshape(NW, C, CHUNK)
    dst_p = dst_p.reshape(NW, C, CHUNK)

    x_pad = jnp.concatenate([x, jnp.zeros((NP - N, D), _F32)])

    r1 = lambda v: v.reshape(1, D)

    sc_agg = _make_sc_agg(C)
    sc_deg = _make_sc_deg(C)

    m0, z0 = _tc_in(x_pad, Wl0, Wr0, r1(bl0))
    (deg,) = sc_deg(dst_p)
    (acc,) = sc_agg(m0, src_p, dst_p)
    m1, z1 = _tc_mid(acc[0], acc[1], deg[0], deg[1], z0,
                     r1(g0), r1(be0), r1(rm0), r1(rv0), Wl1, Wr1, r1(bl1))
    (acc,) = sc_agg(m1, src_p, dst_p)
    m2, z2 = _tc_mid(acc[0], acc[1], deg[0], deg[1], z1,
                     r1(g1), r1(be1), r1(rm1), r1(rv1), Wl2, Wr2, r1(bl2))
    (acc,) = sc_agg(m2, src_p, dst_p)
    out = _tc_out(acc[0], acc[1], deg[0], deg[1], z2)
    return out[:N]
